# baseline (speedup 1.0000x reference)
import functools

import jax
import jax.numpy as jnp
from jax import lax
from jax.experimental import pallas as pl
from jax.experimental.pallas import tpu as pltpu
from jax.experimental.pallas import tpu_sc as plsc

_NC = 2
_NS = 16
_NW = _NC * _NS
_L = 16

_ROWS = 2048
_COLS = 2048
_ROWS_PER_W = _ROWS // _NW
_CR = 8
_NCHUNKS = _ROWS_PER_W // _CR
_NBUF = 6
_A = 4


def _remap_sc(label, mapping):
    mesh = plsc.VectorSubcoreMesh(core_axis_name="c", subcore_axis_name="s")

    @functools.partial(
        pl.kernel,
        out_type=jax.ShapeDtypeStruct((_ROWS, _COLS), jnp.int32),
        mesh=mesh,
        compiler_params=pltpu.CompilerParams(needs_layout_passes=False),
        scratch_types=[
            pltpu.VMEM((128,), jnp.int32),
            pltpu.VMEM((_NBUF * _CR, _COLS), jnp.int32),
            pltpu.SemaphoreType.DMA((_NBUF,)),
            pltpu.SemaphoreType.DMA((_NBUF,)),
        ],
    )
    def k(label_hbm, map_hbm, out_hbm, table_v, ring, isems, osems):
        wid = lax.axis_index("s") * _NC + lax.axis_index("c")

        pltpu.sync_copy(map_hbm, table_v.at[pl.ds(0, 34)])

        def in_copy(ci):
            b = ci % _NBUF
            row0 = wid * _ROWS_PER_W + ci * _CR
            return pltpu.make_async_copy(
                label_hbm.at[pl.ds(row0, _CR), :],
                ring.at[pl.ds(b * _CR, _CR), :], isems.at[b])

        def out_copy(ci):
            b = ci % _NBUF
            row0 = wid * _ROWS_PER_W + ci * _CR
            return pltpu.make_async_copy(
                ring.at[pl.ds(b * _CR, _CR), :],
                out_hbm.at[pl.ds(row0, _CR), :], osems.at[b])

        for ci in range(min(_A, _NCHUNKS)):
            in_copy(ci).start()

        for ci in range(_NCHUNKS):
            b = ci % _NBUF
            in_copy(ci).wait()

            for r in range(_CR):
                @plsc.parallel_loop(0, _COLS, _L, unroll=8)
                def _(i):
                    idx = ring[b * _CR + r, pl.ds(i, _L)]
                    ring[b * _CR + r, pl.ds(i, _L)] = (
                        plsc.load_gather(table_v, [idx]))

            out_copy(ci).start()
            nxt = ci + _A
            if nxt < _NCHUNKS:
                if nxt >= _NBUF:
                    out_copy(nxt - _NBUF).wait()
                in_copy(nxt).start()

        for ci in range(max(0, _NCHUNKS - _NBUF), _NCHUNKS):
            out_copy(ci).wait()

    return k(label, mapping)


def _copy_body(x_ref, o_ref):
    o_ref[...] = x_ref[...]


def _image_copy_tc(image):
    c, h, w = image.shape
    bh = 256
    return pl.pallas_call(
        _copy_body,
        grid=(c, h // bh),
        in_specs=[pl.BlockSpec((1, bh, w), lambda i, j: (i, j, 0))],
        out_specs=pl.BlockSpec((1, bh, w), lambda i, j: (i, j, 0)),
        out_shape=jax.ShapeDtypeStruct(image.shape, image.dtype),
    )(image)


def kernel(image, label, mapping):
    image_out = _image_copy_tc(image)
    remapped = _remap_sc(label.astype(jnp.int32), mapping.astype(jnp.int32))
    return (image_out, remapped.astype(mapping.dtype))


# speedup vs baseline: 1.0659x; 1.0659x over previous
import functools

import jax
import jax.numpy as jnp
from jax import lax
from jax.experimental import pallas as pl
from jax.experimental.pallas import tpu as pltpu
from jax.experimental.pallas import tpu_sc as plsc

_NC = 2
_NS = 16
_NW = _NC * _NS
_L = 16

_ROWS = 2048
_COLS = 2048
_ROWS_PER_W = _ROWS // _NW
_CR = 8
_NCHUNKS = _ROWS_PER_W // _CR
_NBUF = 6
_A = 4


def _remap_sc(label, mapping):
    mesh = plsc.VectorSubcoreMesh(core_axis_name="c", subcore_axis_name="s")

    @functools.partial(
        pl.kernel,
        out_type=jax.ShapeDtypeStruct((_ROWS, _COLS), jnp.int32),
        mesh=mesh,
        compiler_params=pltpu.CompilerParams(needs_layout_passes=False),
        scratch_types=[
            pltpu.VMEM((128,), jnp.int32),
            pltpu.VMEM((_NBUF * _CR, _COLS), jnp.int32),
            pltpu.SemaphoreType.DMA((_NBUF,)),
            pltpu.SemaphoreType.DMA((_NBUF,)),
        ],
    )
    def k(label_hbm, map_hbm, out_hbm, table_v, ring, isems, osems):
        wid = lax.axis_index("s") * _NC + lax.axis_index("c")

        pltpu.sync_copy(map_hbm, table_v.at[pl.ds(0, 34)])

        def in_copy(ci):
            b = ci % _NBUF
            row0 = wid * _ROWS_PER_W + ci * _CR
            return pltpu.make_async_copy(
                label_hbm.at[pl.ds(row0, _CR), :],
                ring.at[pl.ds(b * _CR, _CR), :], isems.at[b])

        def out_copy(ci):
            b = ci % _NBUF
            row0 = wid * _ROWS_PER_W + ci * _CR
            return pltpu.make_async_copy(
                ring.at[pl.ds(b * _CR, _CR), :],
                out_hbm.at[pl.ds(row0, _CR), :], osems.at[b])

        for ci in range(min(_A, _NCHUNKS)):
            in_copy(ci).start()

        for ci in range(_NCHUNKS):
            b = ci % _NBUF
            in_copy(ci).wait()

            for r in range(_CR):
                @plsc.parallel_loop(0, _COLS, _L, unroll=8)
                def _(i):
                    idx = ring[b * _CR + r, pl.ds(i, _L)]
                    ring[b * _CR + r, pl.ds(i, _L)] = (
                        plsc.load_gather(table_v, [idx]))

            out_copy(ci).start()
            nxt = ci + _A
            if nxt < _NCHUNKS:
                if nxt >= _NBUF:
                    out_copy(nxt - _NBUF).wait()
                in_copy(nxt).start()

        for ci in range(max(0, _NCHUNKS - _NBUF), _NCHUNKS):
            out_copy(ci).wait()

    return k(label, mapping)


def _copy_body(x_ref, o_ref):
    o_ref[...] = x_ref[...]


def _image_copy_tc(image):
    c, h, w = image.shape
    bh = 1024
    return pl.pallas_call(
        _copy_body,
        grid=(c, h // bh),
        in_specs=[pl.BlockSpec((1, bh, w), lambda i, j: (i, j, 0))],
        out_specs=pl.BlockSpec((1, bh, w), lambda i, j: (i, j, 0)),
        out_shape=jax.ShapeDtypeStruct(image.shape, image.dtype),
    )(image)


def kernel(image, label, mapping):
    image_out = _image_copy_tc(image)
    remapped = _remap_sc(label.astype(jnp.int32), mapping.astype(jnp.int32))
    return (image_out, remapped.astype(mapping.dtype))


# speedup vs baseline: 1.0696x; 1.0035x over previous
import functools

import jax
import jax.numpy as jnp
from jax import lax
from jax.experimental import pallas as pl
from jax.experimental.pallas import tpu as pltpu
from jax.experimental.pallas import tpu_sc as plsc

_NC = 2
_NS = 16
_NW = _NC * _NS
_L = 16

_ROWS = 2048
_COLS = 2048
_ROWS_PER_W = _ROWS // _NW
_CR = 8
_NCHUNKS = _ROWS_PER_W // _CR
_NBUF = 6
_A = 4


def _remap_sc(label, mapping):
    mesh = plsc.VectorSubcoreMesh(core_axis_name="c", subcore_axis_name="s")

    @functools.partial(
        pl.kernel,
        out_type=jax.ShapeDtypeStruct((_ROWS, _COLS), jnp.int32),
        mesh=mesh,
        compiler_params=pltpu.CompilerParams(needs_layout_passes=False),
        scratch_types=[
            pltpu.VMEM((128,), jnp.int32),
            pltpu.VMEM((_NBUF * _CR, _COLS), jnp.int32),
            pltpu.SemaphoreType.DMA((_NBUF,)),
            pltpu.SemaphoreType.DMA((_NBUF,)),
        ],
    )
    def k(label_hbm, map_hbm, out_hbm, table_v, ring, isems, osems):
        wid = lax.axis_index("s") * _NC + lax.axis_index("c")

        def in_copy(ci):
            b = ci % _NBUF
            row0 = wid * _ROWS_PER_W + ci * _CR
            return pltpu.make_async_copy(
                label_hbm.at[pl.ds(row0, _CR), :],
                ring.at[pl.ds(b * _CR, _CR), :], isems.at[b])

        def out_copy(ci):
            b = ci % _NBUF
            row0 = wid * _ROWS_PER_W + ci * _CR
            return pltpu.make_async_copy(
                ring.at[pl.ds(b * _CR, _CR), :],
                out_hbm.at[pl.ds(row0, _CR), :], osems.at[b])

        for ci in range(min(_A, _NCHUNKS)):
            in_copy(ci).start()
        pltpu.sync_copy(map_hbm, table_v.at[pl.ds(0, 34)])

        for ci in range(_NCHUNKS):
            b = ci % _NBUF
            in_copy(ci).wait()

            for r in range(_CR):
                @plsc.parallel_loop(0, _COLS, _L, unroll=8)
                def _(i):
                    idx = ring[b * _CR + r, pl.ds(i, _L)]
                    ring[b * _CR + r, pl.ds(i, _L)] = (
                        plsc.load_gather(table_v, [idx]))

            out_copy(ci).start()
            nxt = ci + _A
            if nxt < _NCHUNKS:
                if nxt >= _NBUF:
                    out_copy(nxt - _NBUF).wait()
                in_copy(nxt).start()

        for ci in range(max(0, _NCHUNKS - _NBUF), _NCHUNKS):
            out_copy(ci).wait()

    return k(label, mapping)


def _copy_body(x_ref, o_ref):
    o_ref[...] = x_ref[...]


def _image_copy_tc(image):
    c, h, w = image.shape
    bh = 1024
    return pl.pallas_call(
        _copy_body,
        grid=(c, h // bh),
        in_specs=[pl.BlockSpec((1, bh, w), lambda i, j: (i, j, 0))],
        out_specs=pl.BlockSpec((1, bh, w), lambda i, j: (i, j, 0)),
        out_shape=jax.ShapeDtypeStruct(image.shape, image.dtype),
    )(image)


def kernel(image, label, mapping):
    image_out = _image_copy_tc(image)
    remapped = _remap_sc(label.astype(jnp.int32), mapping.astype(jnp.int32))
    return (image_out, remapped.astype(mapping.dtype))
